# Initial kernel scaffold; baseline (speedup 1.0000x reference)
import jax
import jax.numpy as jnp
from jax.experimental import pallas as pl


def kernel(x, w1, b1, g1, beta1, w2, b2, g2, beta2, w3, b3, g3, beta3, w4, b4, g4, beta4):
    raise NotImplementedError("write your pallas kernel here")


# speedup vs baseline: 1.2169x; 1.2169x over previous
import jax
import jax.numpy as jnp
from jax.experimental import pallas as pl
from jax.experimental.pallas import tpu as pltpu

_EPS = 1e-5


def _conv_stats_body(t_ref, w_ref, b_ref, o_ref, s_ref, q_ref):
    acc = jnp.dot(t_ref[...], w_ref[...], preferred_element_type=jnp.float32)
    y = jnp.maximum(acc + b_ref[...], 0.0)
    o_ref[...] = y
    s_ref[...] = jnp.sum(y, axis=0).reshape(1, 1, -1)
    q_ref[...] = jnp.sum(y * y, axis=0).reshape(1, 1, -1)


def _im2col_s2(x):
    n, h, w, cin = x.shape
    ho, wo = h // 2, w // 2
    xp = jnp.pad(x, ((0, 0), (1, 1), (1, 1), (0, 0)))
    slabs = []
    for kh in range(3):
        for kw in range(3):
            slabs.append(xp[:, kh:kh + 2 * ho - 1:2, kw:kw + 2 * wo - 1:2, :])
    pat = jnp.concatenate(slabs, axis=-1)
    return pat.reshape(n * ho * wo, 9 * cin), ho, wo


def _conv_bn_layer(x_bf, w, b, g, beta):
    n = x_bf.shape[0]
    taps, ho, wo = _im2col_s2(x_bf)
    m, k = taps.shape
    cout = w.shape[3]
    wk = w.reshape(k, cout).astype(jnp.bfloat16)
    tm = min(512, m)
    grid = m // tm

    flops = 2 * m * k * cout
    bytes_accessed = m * k * 2 + k * cout * 2 + m * cout * 4

    out, ps, pq = pl.pallas_call(
        _conv_stats_body,
        out_shape=(
            jax.ShapeDtypeStruct((m, cout), jnp.float32),
            jax.ShapeDtypeStruct((grid, 1, cout), jnp.float32),
            jax.ShapeDtypeStruct((grid, 1, cout), jnp.float32),
        ),
        grid=(grid,),
        in_specs=[
            pl.BlockSpec((tm, k), lambda i: (i, 0)),
            pl.BlockSpec((k, cout), lambda i: (0, 0)),
            pl.BlockSpec((1, cout), lambda i: (0, 0)),
        ],
        out_specs=(
            pl.BlockSpec((tm, cout), lambda i: (i, 0)),
            pl.BlockSpec((1, 1, cout), lambda i: (i, 0, 0)),
            pl.BlockSpec((1, 1, cout), lambda i: (i, 0, 0)),
        ),
        compiler_params=pltpu.CompilerParams(
            dimension_semantics=("parallel",),
            vmem_limit_bytes=100 * 1024 * 1024,
        ),
        cost_estimate=pl.CostEstimate(
            flops=flops, transcendentals=0, bytes_accessed=bytes_accessed),
    )(taps, wk, b.astype(jnp.float32).reshape(1, cout))

    mean = ps.sum(axis=0).reshape(cout) / m
    var = pq.sum(axis=0).reshape(cout) / m - mean * mean
    scale = g * jax.lax.rsqrt(var + _EPS)
    shift = beta - mean * scale
    xn = out.reshape(n, ho, wo, cout) * scale + shift
    return xn.astype(jnp.bfloat16)


def _axis_idx(out_size, in_size):
    sc = (in_size - 1) / (out_size - 1) if out_size > 1 else 0.0
    f = jnp.arange(out_size, dtype=jnp.float32) * sc
    lo = jnp.clip(jnp.floor(f).astype(jnp.int32), 0, in_size - 1)
    hi = jnp.minimum(lo + 1, in_size - 1)
    return lo, hi, f - lo.astype(jnp.float32)


def _resize_half(y, oh, ow):
    h, w = y.shape[1], y.shape[2]
    h0, h1, th = _axis_idx(oh, h)
    w0, w1, tw = _axis_idx(ow, w)
    th = th[None, :, None, None].astype(y.dtype)
    tw = tw[None, None, :, None].astype(y.dtype)
    r0 = y[:, h0]
    r1 = y[:, h1]
    top = r0[:, :, w0] * (1 - tw) + r0[:, :, w1] * tw
    bot = r1[:, :, w0] * (1 - tw) + r1[:, :, w1] * tw
    return top * (1 - th) + bot * th


def _interp_mat(out_size, in_size):
    lo, hi, t = _axis_idx(out_size, in_size)
    rows = jnp.arange(out_size)
    mat = jnp.zeros((out_size, in_size), jnp.float32)
    mat = mat.at[rows, lo].add(1.0 - t)
    mat = mat.at[rows, hi].add(t)
    return mat


def _upsum_body(r_ref, z_ref, o_ref):
    o_ref[0] = jnp.dot(r_ref[...], z_ref[0],
                       preferred_element_type=jnp.float32)


def _upsample_sum(ys, out_h, out_w):
    n, c = ys[0].shape[0], ys[0].shape[3]
    wc = out_w * c

    r_parts, z_parts = [], []
    for y in ys:
        h, w = y.shape[1], y.shape[2]
        rh = _interp_mat(out_h, h)
        rw = _interp_mat(out_w, w).astype(jnp.bfloat16)
        z = jnp.einsum("ow,nhwc->nhoc", rw, y)
        hp = ((h + 7) // 8) * 8
        r_parts.append(jnp.pad(rh, ((0, 0), (0, hp - h))))
        z_parts.append(jnp.pad(z.reshape(n, h, wc), ((0, 0), (0, hp - h), (0, 0))))

    rcat = jnp.concatenate(r_parts, axis=1)
    zcat = jnp.concatenate(z_parts, axis=1)
    ktot = rcat.shape[1]
    kp = ((ktot + 127) // 128) * 128
    rcat = jnp.pad(rcat, ((0, 0), (0, kp - ktot))).astype(jnp.bfloat16)
    zcat = jnp.pad(zcat, ((0, 0), (0, kp - ktot), (0, 0))).astype(jnp.bfloat16)

    twc = 8192
    grid = (n, wc // twc)

    out = pl.pallas_call(
        _upsum_body,
        out_shape=jax.ShapeDtypeStruct((n, out_h, wc), jnp.float32),
        grid=grid,
        in_specs=[
            pl.BlockSpec((out_h, kp), lambda i, j: (0, 0)),
            pl.BlockSpec((1, kp, twc), lambda i, j: (i, 0, j)),
        ],
        out_specs=pl.BlockSpec((1, out_h, twc), lambda i, j: (i, 0, j)),
        compiler_params=pltpu.CompilerParams(
            dimension_semantics=("parallel", "parallel"),
            vmem_limit_bytes=100 * 1024 * 1024,
        ),
        cost_estimate=pl.CostEstimate(
            flops=2 * n * out_h * kp * wc,
            transcendentals=0,
            bytes_accessed=n * kp * wc * 2 + n * out_h * wc * 4),
    )(rcat, zcat)
    return out.reshape(n, out_h, out_w, c)


def kernel(x, w1, b1, g1, beta1, w2, b2, g2, beta2,
           w3, b3, g3, beta3, w4, b4, g4, beta4):
    xh = jnp.transpose(x, (0, 2, 3, 1)).astype(jnp.bfloat16)
    h, w = xh.shape[1], xh.shape[2]

    x1 = _conv_bn_layer(xh, w1, b1, g1, beta1)
    y1 = x1
    x2 = _conv_bn_layer(x1, w2, b2, g2, beta2)
    y2 = _resize_half(y1, x2.shape[1], x2.shape[2]) + x2
    x3 = _conv_bn_layer(x2, w3, b3, g3, beta3)
    y3 = _resize_half(y2, x3.shape[1], x3.shape[2]) + x3
    x4 = _conv_bn_layer(x3, w4, b4, g4, beta4)
    y4 = _resize_half(y3, x4.shape[1], x4.shape[2]) + x4

    agg = _upsample_sum([y1, y2, y3, y4], h, w)
    return jnp.transpose(agg, (0, 3, 1, 2))
